# Initial kernel scaffold; baseline (speedup 1.0000x reference)
import jax
import jax.numpy as jnp
from jax.experimental import pallas as pl


def kernel(topic_vectors, num_sents, paragraphs, sentence_lengths, max_length, emb_table, W_ih, W_hh, b_ih, b_hh, lin_W, lin_b):
    raise NotImplementedError("write your pallas kernel here")


# speedup vs baseline: 4.1252x; 4.1252x over previous
import functools

import jax
import jax.numpy as jnp
from jax import lax
from jax.experimental import pallas as pl
from jax.experimental.pallas import tpu as pltpu
from jax.experimental.pallas import tpu_sc as plsc

_pc = pl.pallas_call


def _prep_body(nsc_ref, nsr_ref, ml_ref, tv_ref, par_ref, sl_ref,
               sortns_ref, tvs_ref, pars_ref, dec_ref, nsteps_ref):
    bsz = nsc_ref.shape[0]
    S = sl_ref.shape[1]
    nsc = nsc_ref[...]
    nsr = nsr_ref[...]
    ri = lax.broadcasted_iota(jnp.int32, (bsz, bsz), 0)
    ci = lax.broadcasted_iota(jnp.int32, (bsz, bsz), 1)
    before = (nsc > nsr) | ((nsc == nsr) & (ri < ci))
    rank = jnp.sum(before.astype(jnp.int32), axis=0, keepdims=True)
    Ei = (ri == rank).astype(jnp.int32)
    sortns_ref[...] = jnp.sum(Ei * ci, axis=1, keepdims=True)
    ns_s = jnp.sum(Ei * nsr, axis=1, keepdims=True)
    Ef = Ei.astype(jnp.float32)
    tvs_ref[...] = jnp.sum(Ef[:, :, None] * tv_ref[...][None, :, :], axis=1)
    pars_ref[...] = jnp.sum(Ei[:, :, None] * par_ref[...][None, :, :], axis=1)
    sl_s = jnp.sum(Ei[:, :, None] * sl_ref[...][None, :, :], axis=1)
    dec = jnp.maximum(sl_s - 1, 0)
    dec_ref[...] = dec
    si = lax.broadcasted_iota(jnp.int32, (bsz, S), 1)
    ml = ml_ref[...]
    nsteps_ref[...] = jnp.where(ns_s > si, jnp.minimum(dec, ml - 1), 0)


def _recur_body(tv_ref, emb_ref, wih_ref, whh_ref, bih_ref, bhh_ref,
                h2_ref, h_ref, c_ref):
    t = pl.program_id(0)
    H = whh_ref.shape[0]

    @pl.when(t == 0)
    def _():
        h_ref[...] = jnp.zeros_like(h_ref)
        c_ref[...] = jnp.zeros_like(c_ref)

    x = jnp.where(t == 0, tv_ref[...], emb_ref[0])
    gates = jnp.dot(x, wih_ref[...], preferred_element_type=jnp.float32)
    gates += jnp.dot(h_ref[...], whh_ref[...], preferred_element_type=jnp.float32)
    gates += bih_ref[...] + bhh_ref[...]
    i = jax.nn.sigmoid(gates[:, 0:H])
    f = jax.nn.sigmoid(gates[:, H:2 * H])
    g = jnp.tanh(gates[:, 2 * H:3 * H])
    o = jax.nn.sigmoid(gates[:, 3 * H:4 * H])
    c = f * c_ref[...] + i * g
    h = o * jnp.tanh(c)
    c_ref[...] = c
    h_ref[...] = h
    h2_ref[0] = h


def _proj_body(h2_ref, lwt_ref, lb_ref, nsteps_ref, out_ref):
    j = pl.program_id(1)
    p = jnp.dot(h2_ref[0], lwt_ref[...], preferred_element_type=jnp.float32)
    p = p + lb_ref[...]
    act = nsteps_ref[...] > j
    out_ref[...] = jnp.where(act, p, 0.0)[:, None, None, :]


def _sc_gather(table, idx):
    N, D = idx.shape[0], table.shape[1]
    info = plsc.get_sparse_core_info()
    nw = info.num_cores * info.num_subcores
    bpw = N // nw
    mesh = plsc.VectorSubcoreMesh(core_axis_name="c", subcore_axis_name="s")

    @functools.partial(
        pl.kernel, mesh=mesh,
        out_type=jax.ShapeDtypeStruct((N, D), jnp.float32),
        scratch_types=[
            pltpu.VMEM((bpw,), jnp.int32),
            pltpu.VMEM((bpw, D), jnp.float32),
            pltpu.SemaphoreType.DMA,
        ],
    )
    def gather_k(table_hbm, idx_hbm, out_hbm, idx_v, rows_v, sem):
        wid = lax.axis_index("s") * info.num_cores + lax.axis_index("c")
        base = wid * bpw
        pltpu.sync_copy(idx_hbm.at[pl.ds(base, bpw)], idx_v)
        pltpu.async_copy(table_hbm.at[idx_v], rows_v, sem).wait()
        pltpu.sync_copy(rows_v, out_hbm.at[pl.ds(base, bpw)])

    return gather_k(table, idx)


def kernel(topic_vectors, num_sents, paragraphs, sentence_lengths, max_length,
           emb_table, W_ih, W_hh, b_ih, b_hh, lin_W, lin_b):
    B, S, D = topic_vectors.shape
    T = paragraphs.shape[2]
    H = W_hh.shape[1]
    V = lin_W.shape[0]
    R = B * S

    ns = num_sents.astype(jnp.int32)
    f32, i32 = jnp.float32, jnp.int32

    sortns, tvs, pars, dec, nsteps = _pc(
        _prep_body,
        out_shape=[
            jax.ShapeDtypeStruct((B, 1), i32),
            jax.ShapeDtypeStruct((B, S * D), f32),
            jax.ShapeDtypeStruct((B, S * T), i32),
            jax.ShapeDtypeStruct((B, S), i32),
            jax.ShapeDtypeStruct((B, S), i32),
        ],
    )(ns.reshape(B, 1), ns.reshape(1, B),
      jnp.asarray(max_length, i32).reshape(1, 1),
      topic_vectors.reshape(B, S * D),
      paragraphs.astype(i32).reshape(B, S * T),
      sentence_lengths.astype(i32).reshape(B, S))

    tokens = pars.reshape(R, T).T.reshape(R * T)
    emb = _sc_gather(emb_table, tokens).reshape(T, R, D)

    H2 = _pc(
        _recur_body,
        grid=(T,),
        in_specs=[
            pl.BlockSpec((R, D), lambda t: (0, 0)),
            pl.BlockSpec((1, R, D), lambda t: (jnp.maximum(t - 1, 0), 0, 0)),
            pl.BlockSpec((D, 4 * H), lambda t: (0, 0)),
            pl.BlockSpec((H, 4 * H), lambda t: (0, 0)),
            pl.BlockSpec((1, 4 * H), lambda t: (0, 0)),
            pl.BlockSpec((1, 4 * H), lambda t: (0, 0)),
        ],
        out_specs=pl.BlockSpec((1, R, H), lambda t: (t, 0, 0)),
        out_shape=jax.ShapeDtypeStruct((T, R, H), f32),
        scratch_shapes=[pltpu.VMEM((R, H), f32), pltpu.VMEM((R, H), f32)],
    )(tvs.reshape(R, D), emb, W_ih.T, W_hh.T,
      b_ih.reshape(1, 4 * H), b_hh.reshape(1, 4 * H))

    VT = 1024
    NV = pl.cdiv(V, VT)
    preds = _pc(
        _proj_body,
        grid=(NV, T),
        in_specs=[
            pl.BlockSpec((1, R, H), lambda v, j: (jnp.minimum(j + 1, T - 1), 0, 0)),
            pl.BlockSpec((H, VT), lambda v, j: (0, v)),
            pl.BlockSpec((1, VT), lambda v, j: (0, v)),
            pl.BlockSpec((R, 1), lambda v, j: (0, 0)),
        ],
        out_specs=pl.BlockSpec((R, 1, 1, VT), lambda v, j: (0, j, 0, v)),
        out_shape=jax.ShapeDtypeStruct((R, T, 1, V), f32),
    )(H2, lin_W.T, lin_b.reshape(1, V), nsteps.reshape(R, 1))

    return (preds.reshape(B, S, T, V), pars.reshape(B, S, T), dec,
            sortns.reshape(B))
